# baseline (speedup 1.0000x reference)
import jax
import jax.numpy as jnp
from jax import lax
from jax.experimental import pallas as pl
from jax.experimental.pallas import tpu as pltpu
from jax.experimental.pallas import tpu_sc as plsc

BATCH = 16384
EMBED_DIM = 64
H1 = 128
H2 = 64
BN_EPS = 1e-5

_PAIR_ROWS = 500000
_PAIR_W = 2 * EMBED_DIM

_INFO = plsc.get_sparse_core_info()
_NC = _INFO.num_cores
_NS = _INFO.num_subcores
_NW = _NC * _NS
_ROWS_PER_W = BATCH // _NW
_CHUNK = 128
_NCHUNK = _ROWS_PER_W // _CHUNK
_LANES = 16


def _gather_body(idx_hbm, emb_hbm, x_hbm, idx_v, q_v, rows_v, sem):
    wid = lax.axis_index("s") * _NC + lax.axis_index("c")
    base = wid * _ROWS_PER_W
    pltpu.sync_copy(idx_hbm.at[pl.ds(base, _ROWS_PER_W)], idx_v)

    def to_rows(j, carry):
        v = idx_v[pl.ds(j * _LANES, _LANES)]
        q_v[pl.ds(j * _LANES, _LANES)] = lax.shift_right_logical(v, 1)
        return carry

    lax.fori_loop(0, _ROWS_PER_W // _LANES, to_rows, 0)

    for k in range(_NCHUNK):
        pltpu.async_copy(
            emb_hbm.at[q_v.at[pl.ds(k * _CHUNK, _CHUNK)]],
            rows_v.at[pl.ds(k * _CHUNK, _CHUNK), :],
            sem,
        )
    for k in range(_NCHUNK):
        pltpu.make_async_copy(
            emb_hbm.at[q_v.at[pl.ds(k * _CHUNK, _CHUNK)]],
            rows_v.at[pl.ds(k * _CHUNK, _CHUNK), :],
            sem,
        ).wait()
    pltpu.sync_copy(rows_v, x_hbm.at[pl.ds(base, _ROWS_PER_W), :])


_gather = pl.kernel(
    _gather_body,
    out_type=jax.ShapeDtypeStruct((BATCH, _PAIR_W), jnp.float32),
    mesh=plsc.VectorSubcoreMesh(core_axis_name="c", subcore_axis_name="s"),
    scratch_types=[
        pltpu.VMEM((_ROWS_PER_W,), jnp.int32),
        pltpu.VMEM((_ROWS_PER_W,), jnp.int32),
        pltpu.VMEM((_ROWS_PER_W, _PAIR_W), jnp.float32),
        pltpu.SemaphoreType.DMA,
    ],
)


_BLKM = 1024
_INV = 1.0 / (1.0 + BN_EPS) ** 0.5


def _mlp_body(x2_ref, par_ref, w1_ref, b1_ref, g1_ref, be1_ref, w2_ref,
              b2_ref, g2_ref, be2_ref, o_ref):
    x2 = x2_ref[...]
    odd = par_ref[...] == 1
    x = jnp.where(odd, x2[:, EMBED_DIM:], x2[:, :EMBED_DIM])
    h = jnp.dot(x, w1_ref[...], preferred_element_type=jnp.float32)
    h = h + b1_ref[...]
    h = jnp.maximum(h, 0.0)
    h = h * (_INV * g1_ref[...]) + be1_ref[...]
    o = jnp.dot(h, w2_ref[...], preferred_element_type=jnp.float32)
    o = o + b2_ref[...]
    o = jnp.maximum(o, 0.0)
    o_ref[...] = o * (_INV * g2_ref[...]) + be2_ref[...]


def _full(shape):
    return pl.BlockSpec(shape, lambda i: (0,) * len(shape))


_mlp = pl.pallas_call(
    _mlp_body,
    grid=(BATCH // _BLKM,),
    in_specs=[
        pl.BlockSpec((_BLKM, _PAIR_W), lambda i: (i, 0)),
        pl.BlockSpec((_BLKM, 1), lambda i: (i, 0)),
        _full((EMBED_DIM, H1)),
        _full((1, H1)),
        _full((1, H1)),
        _full((1, H1)),
        _full((H1, H2)),
        _full((1, H2)),
        _full((1, H2)),
        _full((1, H2)),
    ],
    out_specs=pl.BlockSpec((_BLKM, H2), lambda i: (i, 0)),
    out_shape=jax.ShapeDtypeStruct((BATCH, H2), jnp.float32),
)


@jax.jit
def kernel(user_ids, emb, W1, b1, g1, be1, W2, b2, g2, be2):
    idx = user_ids.astype(jnp.int32)
    emb2 = emb.reshape(_PAIR_ROWS, _PAIR_W)
    x2 = _gather(idx, emb2)
    parity = (idx & 1).reshape(BATCH, 1)
    return _mlp(
        x2,
        parity,
        W1,
        b1.reshape(1, H1),
        g1.reshape(1, H1),
        be1.reshape(1, H1),
        W2,
        b2.reshape(1, H2),
        g2.reshape(1, H2),
        be2.reshape(1, H2),
    )
